# baseline (speedup 1.0000x reference)
import functools

import jax
import jax.numpy as jnp
from jax import lax
from jax.experimental import pallas as pl
from jax.experimental.pallas import tpu as pltpu
from jax.experimental.pallas import tpu_sc as plsc

N = 10000
E = 160000
D = 256
DH = 128
NP_ = 10240
EP = 163840
NS = 16
CHUNK = 128
CH_PER_SUB = EP // (NS * CHUNK)
NPH = 5
CPP = CH_PER_SUB // NPH
ROWS_PER_SUB = NP_ // NS
MMB = 1024
FB = 1000

@functools.cache
def _mesh():
    return plsc.VectorSubcoreMesh(core_axis_name="c", subcore_axis_name="s",
                                  num_cores=2, num_subcores=NS)


def _mm_body(x_ref, deg_ref, w_ref, h2a_ref, h2b_ref):
    dis = lax.rsqrt(deg_ref[...] + 1.0)
    h2 = jnp.dot(x_ref[...], w_ref[...],
                 preferred_element_type=jnp.float32) * dis
    h2a_ref[...] = h2[:, :DH]
    h2b_ref[...] = h2[:, DH:]


def _matmul_scale(x, deg_col, W):
    return pl.pallas_call(
        _mm_body,
        grid=(NP_ // MMB,),
        in_specs=[pl.BlockSpec((MMB, D), lambda i: (i, 0)),
                  pl.BlockSpec((MMB, 1), lambda i: (i, 0)),
                  pl.BlockSpec((D, D), lambda i: (0, 0))],
        out_specs=[pl.BlockSpec((MMB, DH), lambda i: (i, 0)),
                   pl.BlockSpec((MMB, DH), lambda i: (i, 0))],
        out_shape=[jax.ShapeDtypeStruct((NP_, DH), jnp.float32),
                   jax.ShapeDtypeStruct((NP_, DH), jnp.float32)],
    )(x, deg_col, W)


def _deg_sc(dst2):
    @functools.partial(
        pl.kernel,
        out_type=jax.ShapeDtypeStruct((NP_,), jnp.float32),
        mesh=_mesh(),
        scratch_types=[
            pltpu.VMEM((CH_PER_SUB, CHUNK), jnp.int32),
            pltpu.VMEM((CHUNK,), jnp.float32),
            pltpu.VMEM((ROWS_PER_SUB,), jnp.float32),
            pltpu.VMEM_SHARED((NP_,), jnp.float32),
        ],
    )
    def k(dst_hbm, deg_hbm, idx_v, ones_v, zero_v, deg_sp):
        cid = lax.axis_index("c")
        sid = lax.axis_index("s")

        @pl.when(cid == 0)
        def _():
            @pl.loop(0, CHUNK, step=16)
            def _(i):
                ones_v[pl.ds(i, 16)] = jnp.ones((16,), jnp.float32)

            @pl.loop(0, ROWS_PER_SUB, step=16)
            def _(i):
                zero_v[pl.ds(i, 16)] = jnp.zeros((16,), jnp.float32)

            base = sid * ROWS_PER_SUB
            pltpu.sync_copy(zero_v, deg_sp.at[pl.ds(base, ROWS_PER_SUB)])
            plsc.subcore_barrier()
            pltpu.sync_copy(dst_hbm.at[pl.ds(sid * CH_PER_SUB, CH_PER_SUB)],
                            idx_v)

            @pl.loop(0, CH_PER_SUB)
            def _(j):
                pltpu.sync_copy(ones_v, deg_sp.at[idx_v.at[j]], add=True)

            plsc.subcore_barrier()
            pltpu.sync_copy(deg_sp.at[pl.ds(base, ROWS_PER_SUB)],
                            deg_hbm.at[pl.ds(base, ROWS_PER_SUB)])

    return k(dst2)


def _scatter_sc(h2a, h2b, src2, dst2):
    @functools.partial(
        pl.kernel,
        out_type=(jax.ShapeDtypeStruct((NP_, DH), jnp.float32),
                  jax.ShapeDtypeStruct((NP_, DH), jnp.float32)),
        mesh=_mesh(),
        scratch_types=[
            pltpu.VMEM((CPP, CHUNK), jnp.int32),
            pltpu.VMEM((CPP, CHUNK), jnp.int32),
            pltpu.VMEM((CHUNK, DH), jnp.float32),
            pltpu.VMEM((CHUNK, DH), jnp.float32),
            pltpu.VMEM_SHARED((NP_, DH), jnp.float32),
            pltpu.SemaphoreType.DMA,
            pltpu.SemaphoreType.DMA,
        ],
    )
    def k(h2a_hbm, h2b_hbm, src_hbm, dst_hbm, s0_hbm, s1_hbm,
          isrc, idst, buf0, buf1, acc, sem0, sem1):
        cid = lax.axis_index("c")
        sid = lax.axis_index("s")
        base = sid * ROWS_PER_SUB

        def run(h2_hbm, out_hbm):
            pltpu.sync_copy(h2_hbm.at[pl.ds(base, ROWS_PER_SUB)],
                            acc.at[pl.ds(base, ROWS_PER_SUB)])
            plsc.subcore_barrier()

            @pl.loop(0, NPH)
            def _(p):
                pbase = sid * CH_PER_SUB + p * CPP
                pltpu.sync_copy(src_hbm.at[pl.ds(pbase, CPP)], isrc)
                pltpu.sync_copy(dst_hbm.at[pl.ds(pbase, CPP)], idst)
                pltpu.async_copy(h2_hbm.at[isrc.at[0]], buf0, sem0)
                pltpu.async_copy(h2_hbm.at[isrc.at[1]], buf1, sem1)

                @pl.loop(0, CPP, step=2)
                def _(j):
                    pltpu.make_async_copy(h2_hbm.at[isrc.at[j]], buf0,
                                          sem0).wait()
                    pltpu.sync_copy(buf0, acc.at[idst.at[j]], add=True)

                    @pl.when(j + 2 < CPP)
                    def _():
                        pltpu.async_copy(h2_hbm.at[isrc.at[j + 2]], buf0, sem0)

                    pltpu.make_async_copy(h2_hbm.at[isrc.at[j + 1]], buf1,
                                          sem1).wait()
                    pltpu.sync_copy(buf1, acc.at[idst.at[j + 1]], add=True)

                    @pl.when(j + 3 < CPP)
                    def _():
                        pltpu.async_copy(h2_hbm.at[isrc.at[j + 3]], buf1, sem1)

            plsc.subcore_barrier()
            pltpu.sync_copy(acc.at[pl.ds(base, ROWS_PER_SUB)],
                            out_hbm.at[pl.ds(base, ROWS_PER_SUB)])

        @pl.when(cid == 0)
        def _():
            run(h2a_hbm, s0_hbm)

        @pl.when(cid == 1)
        def _():
            run(h2b_hbm, s1_hbm)

    return k(h2a, h2b, src2, dst2)


def _final_body(s0_ref, s1_ref, x_ref, deg_ref, b_ref,
                g_ref, bt_ref, o_ref):
    dis = lax.rsqrt(deg_ref[...] + 1.0)
    pre = jnp.concatenate([s0_ref[...], s1_ref[...]], axis=1)
    pre = pre * dis + b_ref[...]
    mu = jnp.mean(pre, axis=-1, keepdims=True)
    var = jnp.mean((pre - mu) ** 2, axis=-1, keepdims=True)
    y = (pre - mu) / jnp.sqrt(var + 1e-5) * g_ref[...] + bt_ref[...]
    o_ref[...] = y * 0.5 * (1.0 + lax.erf(y * 0.7071067811865476)) + x_ref[...]


def _final(s0, s1, x, deg_col, b, gamma, beta):
    vec = pl.BlockSpec((1, D), lambda i: (0, 0))
    half = pl.BlockSpec((FB, DH), lambda i: (i, 0))
    return pl.pallas_call(
        _final_body,
        grid=(N // FB,),
        in_specs=[half, half,
                  pl.BlockSpec((FB, D), lambda i: (i, 0)),
                  pl.BlockSpec((FB, 1), lambda i: (i, 0)),
                  vec, vec, vec],
        out_specs=pl.BlockSpec((FB, D), lambda i: (i, 0)),
        out_shape=jax.ShapeDtypeStruct((N, D), jnp.float32),
    )(s0, s1, x, deg_col, b, gamma, beta)


def kernel(x, edge_index, W, b, gamma, beta):
    src = edge_index[0].astype(jnp.int32)
    dst = edge_index[1].astype(jnp.int32)
    pad = (jnp.arange(EP - E, dtype=jnp.int32) % (NP_ - N)) + N
    src2 = jnp.concatenate([src, pad]).reshape(EP // CHUNK, CHUNK)
    dst2 = jnp.concatenate([dst, pad]).reshape(EP // CHUNK, CHUNK)

    deg = _deg_sc(dst2)
    deg_col = deg.reshape(NP_, 1)
    xp = jnp.concatenate([x, jnp.zeros((NP_ - N, D), jnp.float32)])
    h2a, h2b = _matmul_scale(xp, deg_col, W)
    s0, s1 = _scatter_sc(h2a, h2b, src2, dst2)
    return _final(s0, s1, x, deg_col, b.reshape(1, D),
                  gamma.reshape(1, D), beta.reshape(1, D))


# speedup vs baseline: 1.0062x; 1.0062x over previous
import functools

import jax
import jax.numpy as jnp
from jax import lax
from jax.experimental import pallas as pl
from jax.experimental.pallas import tpu as pltpu
from jax.experimental.pallas import tpu_sc as plsc

N = 10000
E = 160000
D = 256
DH = 128
NP_ = 10240
EP = 163840
NS = 16
CHUNK = 128
CH_PER_SUB = EP // (NS * CHUNK)
NPH = 5
CPP = CH_PER_SUB // NPH
ROWS_PER_SUB = NP_ // NS
MMB = 1024
FB = 1000

@functools.cache
def _mesh():
    return plsc.VectorSubcoreMesh(core_axis_name="c", subcore_axis_name="s",
                                  num_cores=2, num_subcores=NS)


def _mm_body(x_ref, deg_ref, w_ref, h2a_ref, h2b_ref):
    dis = lax.rsqrt(deg_ref[...] + 1.0)
    h2 = jnp.dot(x_ref[...], w_ref[...],
                 preferred_element_type=jnp.float32) * dis
    h2a_ref[...] = h2[:, :DH]
    h2b_ref[...] = h2[:, DH:]


def _matmul_scale(x, deg_col, W):
    return pl.pallas_call(
        _mm_body,
        grid=(NP_ // MMB,),
        in_specs=[pl.BlockSpec((MMB, D), lambda i: (i, 0)),
                  pl.BlockSpec((MMB, 1), lambda i: (i, 0)),
                  pl.BlockSpec((D, D), lambda i: (0, 0))],
        out_specs=[pl.BlockSpec((MMB, DH), lambda i: (i, 0)),
                   pl.BlockSpec((MMB, DH), lambda i: (i, 0))],
        out_shape=[jax.ShapeDtypeStruct((NP_, DH), jnp.float32),
                   jax.ShapeDtypeStruct((NP_, DH), jnp.float32)],
    )(x, deg_col, W)


def _deg_sc(dst2):
    @functools.partial(
        pl.kernel,
        out_type=jax.ShapeDtypeStruct((NP_,), jnp.float32),
        mesh=_mesh(),
        scratch_types=[
            pltpu.VMEM((CH_PER_SUB, CHUNK), jnp.int32),
            pltpu.VMEM((CHUNK,), jnp.float32),
            pltpu.VMEM((ROWS_PER_SUB,), jnp.float32),
            pltpu.VMEM_SHARED((NP_,), jnp.float32),
        ],
    )
    def k(dst_hbm, deg_hbm, idx_v, ones_v, zero_v, deg_sp):
        cid = lax.axis_index("c")
        sid = lax.axis_index("s")

        @pl.when(cid == 0)
        def _():
            @pl.loop(0, CHUNK, step=16)
            def _(i):
                ones_v[pl.ds(i, 16)] = jnp.ones((16,), jnp.float32)

            @pl.loop(0, ROWS_PER_SUB, step=16)
            def _(i):
                zero_v[pl.ds(i, 16)] = jnp.zeros((16,), jnp.float32)

            base = sid * ROWS_PER_SUB
            pltpu.sync_copy(zero_v, deg_sp.at[pl.ds(base, ROWS_PER_SUB)])
            plsc.subcore_barrier()
            pltpu.sync_copy(dst_hbm.at[pl.ds(sid * CH_PER_SUB, CH_PER_SUB)],
                            idx_v)

            @pl.loop(0, CH_PER_SUB)
            def _(j):
                pltpu.sync_copy(ones_v, deg_sp.at[idx_v.at[j]], add=True)

            plsc.subcore_barrier()
            pltpu.sync_copy(deg_sp.at[pl.ds(base, ROWS_PER_SUB)],
                            deg_hbm.at[pl.ds(base, ROWS_PER_SUB)])

    return k(dst2)


def _scatter_sc(h2a, h2b, src2, dst2):
    @functools.partial(
        pl.kernel,
        out_type=(jax.ShapeDtypeStruct((NP_, DH), jnp.float32),
                  jax.ShapeDtypeStruct((NP_, DH), jnp.float32)),
        mesh=_mesh(),
        scratch_types=[
            pltpu.VMEM((CPP, CHUNK), jnp.int32),
            pltpu.VMEM((CPP, CHUNK), jnp.int32),
            pltpu.VMEM((CHUNK, DH), jnp.float32),
            pltpu.VMEM((CHUNK, DH), jnp.float32),
            pltpu.VMEM_SHARED((NP_, DH), jnp.float32),
            pltpu.SemaphoreType.DMA,
            pltpu.SemaphoreType.DMA,
        ],
    )
    def k(h2a_hbm, h2b_hbm, src_hbm, dst_hbm, s0_hbm, s1_hbm,
          isrc, idst, buf0, buf1, acc, sem0, sem1):
        cid = lax.axis_index("c")
        sid = lax.axis_index("s")
        base = sid * ROWS_PER_SUB

        def run(h2_hbm, out_hbm):
            pltpu.async_copy(h2_hbm.at[pl.ds(base, ROWS_PER_SUB)],
                             acc.at[pl.ds(base, ROWS_PER_SUB)], sem0)
            pltpu.sync_copy(src_hbm.at[pl.ds(sid * CH_PER_SUB, CPP)], isrc)
            pltpu.sync_copy(dst_hbm.at[pl.ds(sid * CH_PER_SUB, CPP)], idst)
            pltpu.make_async_copy(h2_hbm.at[pl.ds(base, ROWS_PER_SUB)],
                                  acc.at[pl.ds(base, ROWS_PER_SUB)],
                                  sem0).wait()
            plsc.subcore_barrier()

            @pl.loop(0, NPH)
            def _(p):
                pbase = sid * CH_PER_SUB + p * CPP

                @pl.when(p > 0)
                def _():
                    pltpu.sync_copy(src_hbm.at[pl.ds(pbase, CPP)], isrc)
                    pltpu.sync_copy(dst_hbm.at[pl.ds(pbase, CPP)], idst)

                pltpu.async_copy(h2_hbm.at[isrc.at[0]], buf0, sem0)
                pltpu.async_copy(h2_hbm.at[isrc.at[1]], buf1, sem1)

                @pl.loop(0, CPP, step=2)
                def _(j):
                    pltpu.make_async_copy(h2_hbm.at[isrc.at[j]], buf0,
                                          sem0).wait()
                    pltpu.sync_copy(buf0, acc.at[idst.at[j]], add=True)

                    @pl.when(j + 2 < CPP)
                    def _():
                        pltpu.async_copy(h2_hbm.at[isrc.at[j + 2]], buf0, sem0)

                    pltpu.make_async_copy(h2_hbm.at[isrc.at[j + 1]], buf1,
                                          sem1).wait()
                    pltpu.sync_copy(buf1, acc.at[idst.at[j + 1]], add=True)

                    @pl.when(j + 3 < CPP)
                    def _():
                        pltpu.async_copy(h2_hbm.at[isrc.at[j + 3]], buf1, sem1)

            plsc.subcore_barrier()
            pltpu.sync_copy(acc.at[pl.ds(base, ROWS_PER_SUB)],
                            out_hbm.at[pl.ds(base, ROWS_PER_SUB)])

        @pl.when(cid == 0)
        def _():
            run(h2a_hbm, s0_hbm)

        @pl.when(cid == 1)
        def _():
            run(h2b_hbm, s1_hbm)

    return k(h2a, h2b, src2, dst2)


def _final_body(s0_ref, s1_ref, x_ref, deg_ref, b_ref,
                g_ref, bt_ref, o_ref):
    dis = lax.rsqrt(deg_ref[...] + 1.0)
    pre = jnp.concatenate([s0_ref[...], s1_ref[...]], axis=1)
    pre = pre * dis + b_ref[...]
    mu = jnp.mean(pre, axis=-1, keepdims=True)
    var = jnp.mean((pre - mu) ** 2, axis=-1, keepdims=True)
    y = (pre - mu) / jnp.sqrt(var + 1e-5) * g_ref[...] + bt_ref[...]
    o_ref[...] = y * 0.5 * (1.0 + lax.erf(y * 0.7071067811865476)) + x_ref[...]


def _final(s0, s1, x, deg_col, b, gamma, beta):
    vec = pl.BlockSpec((1, D), lambda i: (0, 0))
    half = pl.BlockSpec((FB, DH), lambda i: (i, 0))
    return pl.pallas_call(
        _final_body,
        grid=(N // FB,),
        in_specs=[half, half,
                  pl.BlockSpec((FB, D), lambda i: (i, 0)),
                  pl.BlockSpec((FB, 1), lambda i: (i, 0)),
                  vec, vec, vec],
        out_specs=pl.BlockSpec((FB, D), lambda i: (i, 0)),
        out_shape=jax.ShapeDtypeStruct((N, D), jnp.float32),
    )(s0, s1, x, deg_col, b, gamma, beta)


def kernel(x, edge_index, W, b, gamma, beta):
    src = edge_index[0].astype(jnp.int32)
    dst = edge_index[1].astype(jnp.int32)
    pad = (jnp.arange(EP - E, dtype=jnp.int32) % (NP_ - N)) + N
    src2 = jnp.concatenate([src, pad]).reshape(EP // CHUNK, CHUNK)
    dst2 = jnp.concatenate([dst, pad]).reshape(EP // CHUNK, CHUNK)

    deg = _deg_sc(dst2)
    deg_col = deg.reshape(NP_, 1)
    xp = jnp.concatenate([x, jnp.zeros((NP_ - N, D), jnp.float32)])
    h2a, h2b = _matmul_scale(xp, deg_col, W)
    s0, s1 = _scatter_sc(h2a, h2b, src2, dst2)
    return _final(s0, s1, x, deg_col, b.reshape(1, D),
                  gamma.reshape(1, D), beta.reshape(1, D))
